# initial kernel scaffold (unmeasured)
import jax
import jax.numpy as jnp
from jax import lax
from jax.experimental import pallas as pl
from jax.experimental.pallas import tpu as pltpu


def kernel(
    x,
):
    def body(*refs):
        pass

    out_shape = jax.ShapeDtypeStruct(..., jnp.float32)
    return pl.pallas_call(body, out_shape=out_shape)(...)


# baseline (device time: 196552 ns/iter reference)
import jax
import jax.numpy as jnp
from jax import lax
from jax.experimental import pallas as pl
from jax.experimental.pallas import tpu as pltpu


def kernel(x):
    m, n = x.shape

    def body(x_ref, out_ref, recv_ref, send_sem, recv_sem):
        my_x = lax.axis_index("x")
        my_y = lax.axis_index("y")
        my_z = lax.axis_index("z")
        partner = (1 - my_x, my_y, my_z)

        barrier_sem = pltpu.get_barrier_semaphore()
        pl.semaphore_signal(
            barrier_sem, inc=1, device_id=partner,
            device_id_type=pl.DeviceIdType.MESH,
        )
        pl.semaphore_wait(barrier_sem, 1)

        rdma = pltpu.make_async_remote_copy(
            src_ref=x_ref,
            dst_ref=recv_ref,
            send_sem=send_sem,
            recv_sem=recv_sem,
            device_id=partner,
            device_id_type=pl.DeviceIdType.MESH,
        )
        rdma.start()
        rdma.wait()
        out_ref[...] = x_ref[...] + recv_ref[...]

    return pl.pallas_call(
        body,
        out_shape=jax.ShapeDtypeStruct((m, n), x.dtype),
        in_specs=[pl.BlockSpec(memory_space=pltpu.VMEM)],
        out_specs=pl.BlockSpec(memory_space=pltpu.VMEM),
        scratch_shapes=[
            pltpu.VMEM((m, n), x.dtype),
            pltpu.SemaphoreType.DMA,
            pltpu.SemaphoreType.DMA,
        ],
        compiler_params=pltpu.CompilerParams(collective_id=0),
    )(x)


# device time: 109701 ns/iter; 1.7917x vs baseline; 1.7917x over previous
import jax
import jax.numpy as jnp
from jax import lax
from jax.experimental import pallas as pl
from jax.experimental.pallas import tpu as pltpu

M, N = 4096, 1024
NRING = 16
C = 160
DIRECT = M - NRING * C

CYCLE = [
    (0, 0), (1, 0), (2, 0), (3, 0),
    (3, 1), (2, 1), (1, 1), (1, 2),
    (2, 2), (3, 2), (3, 3), (2, 3),
    (1, 3), (0, 3), (0, 2), (0, 1),
]
POS_BY_YZ = [0] * 16
for _p, (_y, _z) in enumerate(CYCLE):
    POS_BY_YZ[_y * 4 + _z] = _p
RIGHT_Y = [CYCLE[(_p + 1) % NRING][0] for _p in range(NRING)]
RIGHT_Z = [CYCLE[(_p + 1) % NRING][1] for _p in range(NRING)]
LEFT_Y = [CYCLE[(_p - 1) % NRING][0] for _p in range(NRING)]
LEFT_Z = [CYCLE[(_p - 1) % NRING][1] for _p in range(NRING)]

CW_HOPS = 8
CCW_HOPS = 7

_MESH = pl.DeviceIdType.MESH


def _lut(idx, table):
    acc = jnp.int32(table[0])
    for k in range(1, len(table)):
        acc = jnp.where(idx == k, jnp.int32(table[k]), acc)
    return acc


def kernel(x):
    def body(
        x_ref,
        out_ref,
        direct_recv,
        chunk_recv,
        ring_buf,
        direct_send_sem,
        direct_recv_sem,
        chunk_send_sem,
        chunk_recv_sem,
        cw_send_sems,
        cw_recv_sems,
        ccw_send_sems,
        ccw_recv_sems,
    ):
        my_x = lax.axis_index("x")
        my_y = lax.axis_index("y")
        my_z = lax.axis_index("z")
        partner = (1 - my_x, my_y, my_z)
        p = _lut(my_y * 4 + my_z, POS_BY_YZ)
        right = (my_x, _lut(p, RIGHT_Y), _lut(p, RIGHT_Z))
        left = (my_x, _lut(p, LEFT_Y), _lut(p, LEFT_Z))

        barrier = pltpu.get_barrier_semaphore()
        for nbr in (partner, right, left):
            pl.semaphore_signal(barrier, inc=1, device_id=nbr,
                                device_id_type=_MESH)
        pl.semaphore_wait(barrier, 3)

        chunk_rdma = pltpu.make_async_remote_copy(
            src_ref=x_ref.at[pl.ds(DIRECT + p * C, C), :],
            dst_ref=chunk_recv,
            send_sem=chunk_send_sem,
            recv_sem=chunk_recv_sem,
            device_id=partner,
            device_id_type=_MESH,
        )
        chunk_rdma.start()
        chunk_rdma.wait()

        direct_rdma = pltpu.make_async_remote_copy(
            src_ref=x_ref.at[pl.ds(0, DIRECT), :],
            dst_ref=direct_recv,
            send_sem=direct_send_sem,
            recv_sem=direct_recv_sem,
            device_id=partner,
            device_id_type=_MESH,
        )
        direct_rdma.start()

        ring_buf[p] = x_ref[pl.ds(DIRECT + p * C, C), :] + chunk_recv[...]
        out_ref[pl.ds(DIRECT + p * C, C), :] = ring_buf[p]

        send_descs = []
        for h in range(CW_HOPS):
            cw_slot = jnp.remainder(p - h, NRING)
            cw_send = pltpu.make_async_remote_copy(
                src_ref=ring_buf.at[cw_slot],
                dst_ref=ring_buf.at[cw_slot],
                send_sem=cw_send_sems.at[h],
                recv_sem=cw_recv_sems.at[h],
                device_id=right,
                device_id_type=_MESH,
            )
            cw_send.start()
            send_descs.append(cw_send)
            if h < CCW_HOPS:
                ccw_slot = jnp.remainder(p + h, NRING)
                ccw_send = pltpu.make_async_remote_copy(
                    src_ref=ring_buf.at[ccw_slot],
                    dst_ref=ring_buf.at[ccw_slot],
                    send_sem=ccw_send_sems.at[h],
                    recv_sem=ccw_recv_sems.at[h],
                    device_id=left,
                    device_id_type=_MESH,
                )
                ccw_send.start()
                send_descs.append(ccw_send)

            o_cw = jnp.remainder(p - h - 1, NRING)
            cw_recv = pltpu.make_async_remote_copy(
                src_ref=ring_buf.at[o_cw],
                dst_ref=ring_buf.at[o_cw],
                send_sem=cw_send_sems.at[h],
                recv_sem=cw_recv_sems.at[h],
                device_id=left,
                device_id_type=_MESH,
            )
            cw_recv.wait_recv()
            out_ref[pl.ds(DIRECT + o_cw * C, C), :] = ring_buf[o_cw]

            if h < CCW_HOPS:
                o_ccw = jnp.remainder(p + h + 1, NRING)
                ccw_recv = pltpu.make_async_remote_copy(
                    src_ref=ring_buf.at[o_ccw],
                    dst_ref=ring_buf.at[o_ccw],
                    send_sem=ccw_send_sems.at[h],
                    recv_sem=ccw_recv_sems.at[h],
                    device_id=right,
                    device_id_type=_MESH,
                )
                ccw_recv.wait_recv()
                out_ref[pl.ds(DIRECT + o_ccw * C, C), :] = ring_buf[o_ccw]

        direct_rdma.wait_recv()
        out_ref[pl.ds(0, DIRECT), :] = (
            x_ref[pl.ds(0, DIRECT), :] + direct_recv[...]
        )
        direct_rdma.wait_send()
        for d in send_descs:
            d.wait_send()

    return pl.pallas_call(
        body,
        out_shape=jax.ShapeDtypeStruct((M, N), x.dtype),
        in_specs=[pl.BlockSpec(memory_space=pltpu.VMEM)],
        out_specs=pl.BlockSpec(memory_space=pltpu.VMEM),
        scratch_shapes=[
            pltpu.VMEM((DIRECT, N), x.dtype),
            pltpu.VMEM((C, N), x.dtype),
            pltpu.VMEM((NRING, C, N), x.dtype),
            pltpu.SemaphoreType.DMA,
            pltpu.SemaphoreType.DMA,
            pltpu.SemaphoreType.DMA,
            pltpu.SemaphoreType.DMA,
            pltpu.SemaphoreType.DMA((CW_HOPS,)),
            pltpu.SemaphoreType.DMA((CW_HOPS,)),
            pltpu.SemaphoreType.DMA((CCW_HOPS,)),
            pltpu.SemaphoreType.DMA((CCW_HOPS,)),
        ],
        compiler_params=pltpu.CompilerParams(
            collective_id=0, vmem_limit_bytes=100 * 1024 * 1024
        ),
    )(x)


# device time: 108946 ns/iter; 1.8041x vs baseline; 1.0069x over previous
import jax
import jax.numpy as jnp
from jax import lax
from jax.experimental import pallas as pl
from jax.experimental.pallas import tpu as pltpu

M, N = 4096, 1024
NRING = 16
C = 160
DIRECT = M - NRING * C

CYCLE = [
    (0, 0), (1, 0), (2, 0), (3, 0),
    (3, 1), (2, 1), (1, 1), (1, 2),
    (2, 2), (3, 2), (3, 3), (2, 3),
    (1, 3), (0, 3), (0, 2), (0, 1),
]
POS_BY_YZ = [0] * 16
for _p, (_y, _z) in enumerate(CYCLE):
    POS_BY_YZ[_y * 4 + _z] = _p
RIGHT_Y = [CYCLE[(_p + 1) % NRING][0] for _p in range(NRING)]
RIGHT_Z = [CYCLE[(_p + 1) % NRING][1] for _p in range(NRING)]
LEFT_Y = [CYCLE[(_p - 1) % NRING][0] for _p in range(NRING)]
LEFT_Z = [CYCLE[(_p - 1) % NRING][1] for _p in range(NRING)]

CW_HOPS = 8
CCW_HOPS = 7

_MESH = pl.DeviceIdType.MESH


def _lut(idx, table):
    acc = jnp.int32(table[0])
    for k in range(1, len(table)):
        acc = jnp.where(idx == k, jnp.int32(table[k]), acc)
    return acc


def kernel(x):
    def body(
        x_ref,
        out_ref,
        direct_recv,
        chunk_recv,
        ring_buf,
        direct_send_sem,
        direct_recv_sem,
        chunk_send_sem,
        chunk_recv_sem,
        cw_send_sems,
        cw_recv_sems,
        ccw_send_sems,
        ccw_recv_sems,
    ):
        my_x = lax.axis_index("x")
        my_y = lax.axis_index("y")
        my_z = lax.axis_index("z")
        partner = (1 - my_x, my_y, my_z)
        p = _lut(my_y * 4 + my_z, POS_BY_YZ)
        right = (my_x, _lut(p, RIGHT_Y), _lut(p, RIGHT_Z))
        left = (my_x, _lut(p, LEFT_Y), _lut(p, LEFT_Z))

        barrier = pltpu.get_barrier_semaphore()
        for nbr in (partner, right, left):
            pl.semaphore_signal(barrier, inc=1, device_id=nbr,
                                device_id_type=_MESH)
        pl.semaphore_wait(barrier, 3)

        chunk_rdma = pltpu.make_async_remote_copy(
            src_ref=x_ref.at[pl.ds(DIRECT + p * C, C), :],
            dst_ref=chunk_recv,
            send_sem=chunk_send_sem,
            recv_sem=chunk_recv_sem,
            device_id=partner,
            device_id_type=_MESH,
        )
        chunk_rdma.start()
        chunk_rdma.wait()

        direct_rdma = pltpu.make_async_remote_copy(
            src_ref=x_ref.at[pl.ds(0, DIRECT), :],
            dst_ref=direct_recv,
            send_sem=direct_send_sem,
            recv_sem=direct_recv_sem,
            device_id=partner,
            device_id_type=_MESH,
        )
        direct_rdma.start()

        ring_buf[p] = x_ref[pl.ds(DIRECT + p * C, C), :] + chunk_recv[...]

        def _cw_send(h):
            slot = jnp.remainder(p - h, NRING)
            d = pltpu.make_async_remote_copy(
                src_ref=ring_buf.at[slot],
                dst_ref=ring_buf.at[slot],
                send_sem=cw_send_sems.at[h],
                recv_sem=cw_recv_sems.at[h],
                device_id=right,
                device_id_type=_MESH,
            )
            d.start()
            return d

        def _ccw_send(h):
            slot = jnp.remainder(p + h, NRING)
            d = pltpu.make_async_remote_copy(
                src_ref=ring_buf.at[slot],
                dst_ref=ring_buf.at[slot],
                send_sem=ccw_send_sems.at[h],
                recv_sem=ccw_recv_sems.at[h],
                device_id=left,
                device_id_type=_MESH,
            )
            d.start()
            return d

        def _wait_recv(origin, recv_sems, h):
            pltpu.make_async_remote_copy(
                src_ref=ring_buf.at[origin],
                dst_ref=ring_buf.at[origin],
                send_sem=cw_send_sems.at[0],
                recv_sem=recv_sems.at[h],
                device_id=left,
                device_id_type=_MESH,
            ).wait_recv()

        send_descs = [_cw_send(0), _ccw_send(0)]
        out_ref[pl.ds(DIRECT + p * C, C), :] = ring_buf[p]
        for h in range(CW_HOPS):
            o_cw = jnp.remainder(p - h - 1, NRING)
            _wait_recv(o_cw, cw_recv_sems, h)
            if h + 1 < CW_HOPS:
                send_descs.append(_cw_send(h + 1))
            o_ccw = None
            if h < CCW_HOPS:
                o_ccw = jnp.remainder(p + h + 1, NRING)
                _wait_recv(o_ccw, ccw_recv_sems, h)
                if h + 1 < CCW_HOPS:
                    send_descs.append(_ccw_send(h + 1))
            out_ref[pl.ds(DIRECT + o_cw * C, C), :] = ring_buf[o_cw]
            if o_ccw is not None:
                out_ref[pl.ds(DIRECT + o_ccw * C, C), :] = ring_buf[o_ccw]

        direct_rdma.wait_recv()
        out_ref[pl.ds(0, DIRECT), :] = (
            x_ref[pl.ds(0, DIRECT), :] + direct_recv[...]
        )
        direct_rdma.wait_send()
        for d in send_descs:
            d.wait_send()

    return pl.pallas_call(
        body,
        out_shape=jax.ShapeDtypeStruct((M, N), x.dtype),
        in_specs=[pl.BlockSpec(memory_space=pltpu.VMEM)],
        out_specs=pl.BlockSpec(memory_space=pltpu.VMEM),
        scratch_shapes=[
            pltpu.VMEM((DIRECT, N), x.dtype),
            pltpu.VMEM((C, N), x.dtype),
            pltpu.VMEM((NRING, C, N), x.dtype),
            pltpu.SemaphoreType.DMA,
            pltpu.SemaphoreType.DMA,
            pltpu.SemaphoreType.DMA,
            pltpu.SemaphoreType.DMA,
            pltpu.SemaphoreType.DMA((CW_HOPS,)),
            pltpu.SemaphoreType.DMA((CW_HOPS,)),
            pltpu.SemaphoreType.DMA((CCW_HOPS,)),
            pltpu.SemaphoreType.DMA((CCW_HOPS,)),
        ],
        compiler_params=pltpu.CompilerParams(
            collective_id=0, vmem_limit_bytes=100 * 1024 * 1024
        ),
    )(x)
